# Initial kernel scaffold; baseline (speedup 1.0000x reference)
import jax
import jax.numpy as jnp
from jax.experimental import pallas as pl


def kernel(pointCloudPose, featureVector, PointCloudNormal, SH, rri_W1, rri_b1, rri_W2, rri_b2, conv_W1, conv_b1, bn_g1, bn_b1, conv_W2, conv_b2, bn_g2, bn_b2):
    raise NotImplementedError("write your pallas kernel here")


# speedup vs baseline: 1.6212x; 1.6212x over previous
import functools

import jax
import jax.numpy as jnp
from jax.experimental import pallas as pl

RATIO = 0.25
KNN = 32


def _fps_body(pos_ref, idx_ref):
    B, _, N = pos_ref.shape
    S = idx_ref.shape[1]
    x = pos_ref[:, 0, :]
    y = pos_ref[:, 1, :]
    z = pos_ref[:, 2, :]
    iota_n = jax.lax.broadcasted_iota(jnp.int32, (B, N), 1)
    col_s = jax.lax.broadcasted_iota(jnp.int32, (B, S), 1)

    def body(i, st):
        dist, lx, ly, lz, acc = st
        dx = x - lx
        dy = y - ly
        dz = z - lz
        d = (dx * dx + dy * dy) + dz * dz
        dist = jnp.minimum(dist, d)
        m = jnp.max(dist, axis=1, keepdims=True)
        newidx = jnp.min(jnp.where(dist == m, iota_n, N), axis=1, keepdims=True)
        pm = iota_n == newidx
        lx = jnp.sum(jnp.where(pm, x, 0.0), axis=1, keepdims=True)
        ly = jnp.sum(jnp.where(pm, y, 0.0), axis=1, keepdims=True)
        lz = jnp.sum(jnp.where(pm, z, 0.0), axis=1, keepdims=True)
        acc = jnp.where(col_s == i, newidx, acc)
        return (dist, lx, ly, lz, acc)

    dist0 = jnp.full((B, N), jnp.inf, dtype=jnp.float32)
    acc0 = jnp.zeros((B, S), dtype=jnp.int32)
    st = jax.lax.fori_loop(
        1, S, body, (dist0, x[:, 0:1], y[:, 0:1], z[:, 0:1], acc0))
    idx_ref[...] = st[4]


def _fps_pallas(pose):
    B, N, _ = pose.shape
    S = int(N * RATIO)
    posT = jnp.transpose(pose, (0, 2, 1))
    return pl.pallas_call(
        _fps_body,
        out_shape=jax.ShapeDtypeStruct((B, S), jnp.int32),
    )(posT)


def _mm_kernel(x_ref, w_ref, o_ref):
    o_ref[...] = jnp.dot(x_ref[...].astype(jnp.bfloat16),
                         w_ref[...].astype(jnp.bfloat16),
                         preferred_element_type=jnp.float32)


def _pallas_mm(x, w):
    M, Cin = x.shape
    Cout = w.shape[1]
    BM = 4096
    return pl.pallas_call(
        _mm_kernel,
        grid=(M // BM,),
        in_specs=[
            pl.BlockSpec((BM, Cin), lambda i: (i, 0)),
            pl.BlockSpec((Cin, Cout), lambda i: (0, 0)),
        ],
        out_specs=pl.BlockSpec((BM, Cout), lambda i: (i, 0)),
        out_shape=jax.ShapeDtypeStruct((M, Cout), jnp.float32),
    )(x, w)


def _angle(a, b):
    cross = jnp.linalg.norm(jnp.cross(a, b), axis=-1)
    dot = jnp.sum(a * b, axis=-1)
    return jnp.arctan2(cross, dot)


def kernel(pointCloudPose, featureVector, PointCloudNormal, SH,
           rri_W1, rri_b1, rri_W2, rri_b2,
           conv_W1, conv_b1, bn_g1, bn_b1,
           conv_W2, conv_b2, bn_g2, bn_b2):
    B, N, _ = pointCloudPose.shape
    S = int(N * RATIO)
    fps_idx = _fps_pallas(pointCloudPose)
    gather = jax.vmap(lambda a, i: a[i])
    new_pose = gather(pointCloudPose, fps_idx)
    new_normal = gather(PointCloudNormal, fps_idx)
    new_sh = gather(SH, fps_idx)
    d2 = jnp.sum((new_pose[:, :, None, :] - pointCloudPose[:, None, :, :]) ** 2, axis=-1)
    _, nn_idx = jax.lax.top_k(-d2, KNN)
    pose_cluster = gather(pointCloudPose, nn_idx)
    feature_cluster = gather(featureVector, nn_idx)
    normal_cluster = gather(PointCloudNormal, nn_idx)
    p1 = new_pose[:, :, None, :]
    n1 = jnp.broadcast_to(new_normal[:, :, None, :], pose_cluster.shape)
    d = pose_cluster - p1
    dist = jnp.linalg.norm(d, axis=-1, keepdims=True)
    dn = d / (dist + 1e-8)
    zero = dist[..., 0] == 0.0
    f1 = jnp.where(zero, 0.0, _angle(n1, dn))
    f2 = jnp.where(zero, 0.0, _angle(normal_cluster, dn))
    f3 = _angle(n1, normal_cluster)
    point_feature = jnp.stack([f1, f2, f3, dist[..., 0]], axis=-1)
    rri_in = jnp.concatenate([point_feature, dist], axis=-1)

    M = B * S * KNN

    def mm(x, W):
        return _pallas_mm(x.reshape(M, x.shape[-1]), W).reshape(B, S, KNN, W.shape[1])

    h = jax.nn.relu(mm(rri_in, rri_W1) + rri_b1)
    feature_rri = jax.nn.relu(mm(h, rri_W2) + rri_b2)
    grouping = jnp.concatenate([feature_cluster, feature_rri], axis=-1)

    def conv_bn_relu(x, W, b, g, bt):
        y = mm(x, W) + b
        mean = jnp.mean(y, axis=(0, 1, 2), keepdims=True)
        var = jnp.var(y, axis=(0, 1, 2), keepdims=True)
        y = (y - mean) / jnp.sqrt(var + 1e-5) * g + bt
        return jax.nn.relu(y)

    x = conv_bn_relu(grouping, conv_W1, conv_b1, bn_g1, bn_b1)
    x = conv_bn_relu(x, conv_W2, conv_b2, bn_g2, bn_b2)
    new_feat = jnp.max(x, axis=2)
    return (new_pose, new_feat, new_normal, new_sh)


# speedup vs baseline: 3.4466x; 2.1260x over previous
import functools

import jax
import jax.numpy as jnp
from jax import lax
from jax.experimental import pallas as pl
from jax.experimental.pallas import tpu as pltpu
from jax.experimental.pallas import tpu_sc as plsc

RATIO = 0.25
KNN = 32


@functools.lru_cache(maxsize=None)
def _sc_gather_build(n_rows, d, nw, ch):
    rows_pw = n_rows // nw
    n_chunks = rows_pw // ch
    mesh = plsc.VectorSubcoreMesh(core_axis_name="c", subcore_axis_name="s")

    @functools.partial(
        pl.kernel,
        out_type=jax.ShapeDtypeStruct((n_rows, d), jnp.float32),
        mesh=mesh,
        scratch_types=[
            pltpu.VMEM((ch,), jnp.int32),
            pltpu.VMEM((ch, d), jnp.float32),
            pltpu.SemaphoreType.DMA,
        ],
    )
    def k(tab_hbm, idx_hbm, out_hbm, idx_v, rows_v, sem):
        wid = lax.axis_index("s") * 2 + lax.axis_index("c")
        base = wid * rows_pw
        for j in range(n_chunks):
            off = base + j * ch
            pltpu.sync_copy(idx_hbm.at[pl.ds(off, ch)], idx_v)
            pltpu.async_copy(tab_hbm.at[idx_v], rows_v, sem).wait()
            pltpu.sync_copy(rows_v, out_hbm.at[pl.ds(off, ch)])

    return k


def _fps_body(pos_ref, idx_ref):
    B, _, N = pos_ref.shape
    S = idx_ref.shape[1]
    x = pos_ref[:, 0, :]
    y = pos_ref[:, 1, :]
    z = pos_ref[:, 2, :]
    iota_n = jax.lax.broadcasted_iota(jnp.int32, (B, N), 1)
    col_s = jax.lax.broadcasted_iota(jnp.int32, (B, S), 1)

    def body(i, st):
        dist, lx, ly, lz, acc = st
        dx = x - lx
        dy = y - ly
        dz = z - lz
        d = (dx * dx + dy * dy) + dz * dz
        dist = jnp.minimum(dist, d)
        m = jnp.max(dist, axis=1, keepdims=True)
        newidx = jnp.min(jnp.where(dist == m, iota_n, N), axis=1, keepdims=True)
        pm = iota_n == newidx
        lx = jnp.sum(jnp.where(pm, x, 0.0), axis=1, keepdims=True)
        ly = jnp.sum(jnp.where(pm, y, 0.0), axis=1, keepdims=True)
        lz = jnp.sum(jnp.where(pm, z, 0.0), axis=1, keepdims=True)
        acc = jnp.where(col_s == i, newidx, acc)
        return (dist, lx, ly, lz, acc)

    dist0 = jnp.full((B, N), jnp.inf, dtype=jnp.float32)
    acc0 = jnp.zeros((B, S), dtype=jnp.int32)
    st = jax.lax.fori_loop(
        1, S, body, (dist0, x[:, 0:1], y[:, 0:1], z[:, 0:1], acc0))
    idx_ref[...] = st[4]


def _fps_pallas(pose):
    B, N, _ = pose.shape
    S = int(N * RATIO)
    posT = jnp.transpose(pose, (0, 2, 1))
    return pl.pallas_call(
        _fps_body,
        out_shape=jax.ShapeDtypeStruct((B, S), jnp.int32),
    )(posT)


def _mm_kernel(x_ref, w_ref, o_ref):
    o_ref[...] = jnp.dot(x_ref[...].astype(jnp.bfloat16),
                         w_ref[...].astype(jnp.bfloat16),
                         preferred_element_type=jnp.float32)


def _pallas_mm(x, w):
    M, Cin = x.shape
    Cout = w.shape[1]
    BM = 4096
    return pl.pallas_call(
        _mm_kernel,
        grid=(M // BM,),
        in_specs=[
            pl.BlockSpec((BM, Cin), lambda i: (i, 0)),
            pl.BlockSpec((Cin, Cout), lambda i: (0, 0)),
        ],
        out_specs=pl.BlockSpec((BM, Cout), lambda i: (i, 0)),
        out_shape=jax.ShapeDtypeStruct((M, Cout), jnp.float32),
    )(x, w)


def _angle(a, b):
    cross = jnp.linalg.norm(jnp.cross(a, b), axis=-1)
    dot = jnp.sum(a * b, axis=-1)
    return jnp.arctan2(cross, dot)


def kernel(pointCloudPose, featureVector, PointCloudNormal, SH,
           rri_W1, rri_b1, rri_W2, rri_b2,
           conv_W1, conv_b1, bn_g1, bn_b1,
           conv_W2, conv_b2, bn_g2, bn_b2):
    B, N, _ = pointCloudPose.shape
    S = int(N * RATIO)
    fps_idx = _fps_pallas(pointCloudPose)

    zc = jnp.zeros((B, N, 128 - 47), jnp.float32)
    tab = jnp.concatenate(
        [featureVector, pointCloudPose, PointCloudNormal, SH, zc], -1
    ).reshape(B * N, 128)
    boff = (jnp.arange(B, dtype=jnp.int32) * N)
    fps_flat = (fps_idx + boff[:, None]).reshape(-1)

    cent_rows = _sc_gather_build(B * S, 128, 32, 128)(tab, fps_flat)
    new_pose = cent_rows[:, 32:35].reshape(B, S, 3)
    new_normal = cent_rows[:, 35:38].reshape(B, S, 3)
    new_sh = cent_rows[:, 38:47].reshape(B, S, 9)

    d2 = jnp.sum((new_pose[:, :, None, :] - pointCloudPose[:, None, :, :]) ** 2, axis=-1)
    _, nn_idx = jax.lax.top_k(-d2, KNN)
    nn_flat = (nn_idx + boff[:, None, None]).reshape(-1)
    cl_rows = _sc_gather_build(B * S * KNN, 128, 32, 512)(tab, nn_flat)
    feature_cluster = cl_rows[:, 0:32].reshape(B, S, KNN, 32)
    pose_cluster = cl_rows[:, 32:35].reshape(B, S, KNN, 3)
    normal_cluster = cl_rows[:, 35:38].reshape(B, S, KNN, 3)
    p1 = new_pose[:, :, None, :]
    n1 = jnp.broadcast_to(new_normal[:, :, None, :], pose_cluster.shape)
    d = pose_cluster - p1
    dist = jnp.linalg.norm(d, axis=-1, keepdims=True)
    dn = d / (dist + 1e-8)
    zero = dist[..., 0] == 0.0
    f1 = jnp.where(zero, 0.0, _angle(n1, dn))
    f2 = jnp.where(zero, 0.0, _angle(normal_cluster, dn))
    f3 = _angle(n1, normal_cluster)
    point_feature = jnp.stack([f1, f2, f3, dist[..., 0]], axis=-1)
    rri_in = jnp.concatenate([point_feature, dist], axis=-1)

    M = B * S * KNN

    def mm(x, W):
        return _pallas_mm(x.reshape(M, x.shape[-1]), W).reshape(B, S, KNN, W.shape[1])

    h = jax.nn.relu(mm(rri_in, rri_W1) + rri_b1)
    feature_rri = jax.nn.relu(mm(h, rri_W2) + rri_b2)
    grouping = jnp.concatenate([feature_cluster, feature_rri], axis=-1)

    def conv_bn_relu(x, W, b, g, bt):
        y = mm(x, W) + b
        mean = jnp.mean(y, axis=(0, 1, 2), keepdims=True)
        var = jnp.var(y, axis=(0, 1, 2), keepdims=True)
        y = (y - mean) / jnp.sqrt(var + 1e-5) * g + bt
        return jax.nn.relu(y)

    x = conv_bn_relu(grouping, conv_W1, conv_b1, bn_g1, bn_b1)
    x = conv_bn_relu(x, conv_W2, conv_b2, bn_g2, bn_b2)
    new_feat = jnp.max(x, axis=2)
    return (new_pose, new_feat, new_normal, new_sh)


# speedup vs baseline: 7.9675x; 2.3117x over previous
import functools

import jax
import jax.numpy as jnp
from jax import lax
from jax.experimental import pallas as pl
from jax.experimental.pallas import tpu as pltpu
from jax.experimental.pallas import tpu_sc as plsc

RATIO = 0.25
KNN = 32


@functools.lru_cache(maxsize=None)
def _sc_gather_build(n_rows, d, nw, ch):
    rows_pw = n_rows // nw
    n_chunks = rows_pw // ch
    mesh = plsc.VectorSubcoreMesh(core_axis_name="c", subcore_axis_name="s")

    @functools.partial(
        pl.kernel,
        out_type=jax.ShapeDtypeStruct((n_rows, d), jnp.float32),
        mesh=mesh,
        scratch_types=[
            pltpu.VMEM((ch,), jnp.int32),
            pltpu.VMEM((ch, d), jnp.float32),
            pltpu.SemaphoreType.DMA,
        ],
    )
    def k(tab_hbm, idx_hbm, out_hbm, idx_v, rows_v, sem):
        wid = lax.axis_index("s") * 2 + lax.axis_index("c")
        base = wid * rows_pw
        for j in range(n_chunks):
            off = base + j * ch
            pltpu.sync_copy(idx_hbm.at[pl.ds(off, ch)], idx_v)
            pltpu.async_copy(tab_hbm.at[idx_v], rows_v, sem).wait()
            pltpu.sync_copy(rows_v, out_hbm.at[pl.ds(off, ch)])

    return k


def _fps_body(pos_ref, idx_ref):
    B, _, N = pos_ref.shape
    S = idx_ref.shape[1]
    x = pos_ref[:, 0, :]
    y = pos_ref[:, 1, :]
    z = pos_ref[:, 2, :]
    iota_n = jax.lax.broadcasted_iota(jnp.int32, (B, N), 1)
    col_s = jax.lax.broadcasted_iota(jnp.int32, (B, S), 1)

    def body(i, st):
        dist, lx, ly, lz, acc = st
        dx = x - lx
        dy = y - ly
        dz = z - lz
        d = (dx * dx + dy * dy) + dz * dz
        dist = jnp.minimum(dist, d)
        m = jnp.max(dist, axis=1, keepdims=True)
        newidx = jnp.min(jnp.where(dist == m, iota_n, N), axis=1, keepdims=True)
        pm = iota_n == newidx
        lx = jnp.sum(jnp.where(pm, x, 0.0), axis=1, keepdims=True)
        ly = jnp.sum(jnp.where(pm, y, 0.0), axis=1, keepdims=True)
        lz = jnp.sum(jnp.where(pm, z, 0.0), axis=1, keepdims=True)
        acc = jnp.where(col_s == i, newidx, acc)
        return (dist, lx, ly, lz, acc)

    dist0 = jnp.full((B, N), jnp.inf, dtype=jnp.float32)
    acc0 = jnp.zeros((B, S), dtype=jnp.int32)
    st = jax.lax.fori_loop(
        1, S, body, (dist0, x[:, 0:1], y[:, 0:1], z[:, 0:1], acc0))
    idx_ref[...] = st[4]


def _fps_pallas(pose):
    B, N, _ = pose.shape
    S = int(N * RATIO)
    posT = jnp.transpose(pose, (0, 2, 1))
    return pl.pallas_call(
        _fps_body,
        out_shape=jax.ShapeDtypeStruct((B, S), jnp.int32),
    )(posT)


def _knn_body(cent_ref, pos_ref, out_ref):
    _, _, SB = cent_ref.shape
    N = pos_ref.shape[2]
    b = pl.program_id(0)
    cx = cent_ref[0, 0, :][:, None]
    cy = cent_ref[0, 1, :][:, None]
    cz = cent_ref[0, 2, :][:, None]
    x = pos_ref[0, 0, :][None, :]
    y = pos_ref[0, 1, :][None, :]
    z = pos_ref[0, 2, :][None, :]
    dx = cx - x
    dy = cy - y
    dz = cz - z
    d2 = (dx * dx + dy * dy) + dz * dz
    iota_n = jax.lax.broadcasted_iota(jnp.int32, (SB, N), 1)
    col_k = jax.lax.broadcasted_iota(jnp.int32, (SB, KNN), 1)
    out = jnp.zeros((SB, KNN), jnp.int32)
    for k in range(KNN):
        m = jnp.min(d2, axis=1, keepdims=True)
        idx = jnp.min(jnp.where(d2 == m, iota_n, N), axis=1, keepdims=True)
        out = jnp.where(col_k == k, idx, out)
        d2 = jnp.where(iota_n == idx, jnp.inf, d2)
    out_ref[0] = out + b * N


def _knn_pallas(new_pose, pose):
    B, S, _ = new_pose.shape
    N = pose.shape[1]
    SB = 512
    centT = jnp.transpose(new_pose, (0, 2, 1))
    posT = jnp.transpose(pose, (0, 2, 1))
    return pl.pallas_call(
        _knn_body,
        grid=(B, S // SB),
        in_specs=[
            pl.BlockSpec((1, 3, SB), lambda b, s: (b, 0, s)),
            pl.BlockSpec((1, 3, N), lambda b, s: (b, 0, 0)),
        ],
        out_specs=pl.BlockSpec((1, SB, KNN), lambda b, s: (b, s, 0)),
        out_shape=jax.ShapeDtypeStruct((B, S, KNN), jnp.int32),
    )(centT, posT)


def _mm_kernel(x_ref, w_ref, o_ref):
    o_ref[...] = jnp.dot(x_ref[...].astype(jnp.bfloat16),
                         w_ref[...].astype(jnp.bfloat16),
                         preferred_element_type=jnp.float32)


def _pallas_mm(x, w):
    M, Cin = x.shape
    Cout = w.shape[1]
    BM = 4096
    return pl.pallas_call(
        _mm_kernel,
        grid=(M // BM,),
        in_specs=[
            pl.BlockSpec((BM, Cin), lambda i: (i, 0)),
            pl.BlockSpec((Cin, Cout), lambda i: (0, 0)),
        ],
        out_specs=pl.BlockSpec((BM, Cout), lambda i: (i, 0)),
        out_shape=jax.ShapeDtypeStruct((M, Cout), jnp.float32),
    )(x, w)


def _angle(a, b):
    cross = jnp.linalg.norm(jnp.cross(a, b), axis=-1)
    dot = jnp.sum(a * b, axis=-1)
    return jnp.arctan2(cross, dot)


def kernel(pointCloudPose, featureVector, PointCloudNormal, SH,
           rri_W1, rri_b1, rri_W2, rri_b2,
           conv_W1, conv_b1, bn_g1, bn_b1,
           conv_W2, conv_b2, bn_g2, bn_b2):
    B, N, _ = pointCloudPose.shape
    S = int(N * RATIO)
    fps_idx = _fps_pallas(pointCloudPose)

    zc = jnp.zeros((B, N, 128 - 47), jnp.float32)
    tab = jnp.concatenate(
        [featureVector, pointCloudPose, PointCloudNormal, SH, zc], -1
    ).reshape(B * N, 128)
    boff = (jnp.arange(B, dtype=jnp.int32) * N)
    fps_flat = (fps_idx + boff[:, None]).reshape(-1)

    cent_rows = _sc_gather_build(B * S, 128, 32, 128)(tab, fps_flat)
    new_pose = cent_rows[:, 32:35].reshape(B, S, 3)
    new_normal = cent_rows[:, 35:38].reshape(B, S, 3)
    new_sh = cent_rows[:, 38:47].reshape(B, S, 9)

    nn_flat = _knn_pallas(new_pose, pointCloudPose).reshape(-1)
    cl_rows = _sc_gather_build(B * S * KNN, 128, 32, 512)(tab, nn_flat)
    feature_cluster = cl_rows[:, 0:32].reshape(B, S, KNN, 32)
    pose_cluster = cl_rows[:, 32:35].reshape(B, S, KNN, 3)
    normal_cluster = cl_rows[:, 35:38].reshape(B, S, KNN, 3)
    p1 = new_pose[:, :, None, :]
    n1 = jnp.broadcast_to(new_normal[:, :, None, :], pose_cluster.shape)
    d = pose_cluster - p1
    dist = jnp.linalg.norm(d, axis=-1, keepdims=True)
    dn = d / (dist + 1e-8)
    zero = dist[..., 0] == 0.0
    f1 = jnp.where(zero, 0.0, _angle(n1, dn))
    f2 = jnp.where(zero, 0.0, _angle(normal_cluster, dn))
    f3 = _angle(n1, normal_cluster)
    point_feature = jnp.stack([f1, f2, f3, dist[..., 0]], axis=-1)
    rri_in = jnp.concatenate([point_feature, dist], axis=-1)

    M = B * S * KNN

    def mm(x, W):
        return _pallas_mm(x.reshape(M, x.shape[-1]), W).reshape(B, S, KNN, W.shape[1])

    h = jax.nn.relu(mm(rri_in, rri_W1) + rri_b1)
    feature_rri = jax.nn.relu(mm(h, rri_W2) + rri_b2)
    grouping = jnp.concatenate([feature_cluster, feature_rri], axis=-1)

    def conv_bn_relu(x, W, b, g, bt):
        y = mm(x, W) + b
        mean = jnp.mean(y, axis=(0, 1, 2), keepdims=True)
        var = jnp.var(y, axis=(0, 1, 2), keepdims=True)
        y = (y - mean) / jnp.sqrt(var + 1e-5) * g + bt
        return jax.nn.relu(y)

    x = conv_bn_relu(grouping, conv_W1, conv_b1, bn_g1, bn_b1)
    x = conv_bn_relu(x, conv_W2, conv_b2, bn_g2, bn_b2)
    new_feat = jnp.max(x, axis=2)
    return (new_pose, new_feat, new_normal, new_sh)
